# baseline (speedup 1.0000x reference)
import functools

import jax
import jax.numpy as jnp
from jax import lax
from jax.experimental import pallas as pl
from jax.experimental.pallas import tpu as pltpu
from jax.experimental.pallas import tpu_sc as plsc

N = 100000
E = 1600000
V = 64
D = 32
H = 64
C = 2
G = 512

NC = 2
NS = 16
FC = 16
NQ = H // FC

TCB = 512
NPAD = 100352
NBLK_TC = NPAD // TCB
ROWS_PER_TILE = NPAD // NS

EBLK = 128
NB = 4
EPAD_BLOCKS = 12544
EPAD = EPAD_BLOCKS * EBLK
BLK_PER_TILE = EPAD_BLOCKS // NS
HALF_BLOCKS = EPAD_BLOCKS // NC
BLK_PER_TILE_HALF = HALF_BLOCKS // NS
DUMP_ROWS = 392


def _sc_mesh():
    return plsc.VectorSubcoreMesh(
        core_axis_name="c", subcore_axis_name="s",
        num_cores=NC, num_subcores=NS)


def _fill_rows(ref, nrows, value):
    v = jnp.full((FC,), value, jnp.float32)

    @pl.loop(0, nrows)
    def _(i):
        ref[i] = v


def _zero_slab(slab, zbuf, sid):
    base = sid * ROWS_PER_TILE

    @pl.loop(0, ROWS_PER_TILE // EBLK)
    def _(i):
        pltpu.sync_copy(zbuf, slab.at[pl.ds(base + i * EBLK, EBLK)])


def _dump_slab(slab, out_hbm, rows_v, sid):
    base = sid * ROWS_PER_TILE

    @pl.loop(0, ROWS_PER_TILE // DUMP_ROWS)
    def _(i):
        r0 = base + i * DUMP_ROWS
        tmp = rows_v.at[pl.ds(0, DUMP_ROWS)]
        pltpu.sync_copy(slab.at[pl.ds(r0, DUMP_ROWS)], tmp)
        pltpu.sync_copy(tmp, out_hbm.at[pl.ds(r0, DUMP_ROWS)])


def _ones_pass(dst_hbm, slab, idxd_v, ones_v, ssem, blk_lo, nblk_tile, sid):
    base_blk = blk_lo + sid * nblk_tile

    @pl.loop(0, nblk_tile // (2 * NB))
    def _(it):
        for b in range(2):
            b0 = base_blk + it * 2 * NB + b * NB
            pltpu.sync_copy(dst_hbm.at[pl.ds(b0, NB)], idxd_v[b])
            descs = []
            for j in range(NB):
                descs.append(
                    pltpu.async_copy(ones_v, slab.at[idxd_v[b].at[j]],
                                     ssem[b], add=True))
            for d in descs:
                d.wait()


def _gather_pass(src_hbm, dst_hbm, table_hbm, slab,
                 idxs_v, idxd_v, rows_v, gsem, ssem, sid):
    base_blk = sid * BLK_PER_TILE

    def fire_gathers(b0, b):
        pltpu.sync_copy(src_hbm.at[pl.ds(b0, NB)], idxs_v[b])
        pltpu.sync_copy(dst_hbm.at[pl.ds(b0, NB)], idxd_v[b])
        descs = []
        for j in range(NB):
            descs.append(
                pltpu.async_copy(
                    table_hbm.at[idxs_v[b].at[j]],
                    rows_v[b].at[pl.ds(j * EBLK, EBLK)], gsem[b]))
        return descs

    def fire_scatters(b):
        descs = []
        for j in range(NB):
            descs.append(
                pltpu.async_copy(rows_v[b].at[pl.ds(j * EBLK, EBLK)],
                                 slab.at[idxd_v[b].at[j]], ssem[b],
                                 add=True))
        return descs

    @pl.loop(0, BLK_PER_TILE // (2 * NB))
    def _(it):
        b0 = base_blk + it * 2 * NB
        g0 = fire_gathers(b0, 0)
        g1 = fire_gathers(b0 + NB, 1)
        for d in g0:
            d.wait()
        s0 = fire_scatters(0)
        for d in g1:
            d.wait()
        s1 = fire_scatters(1)
        for d in s0:
            d.wait()
        for d in s1:
            d.wait()


def _sc1_body(src_hbm, dst_hbm, t0, t1, t2, t3,
              deg0, deg1, a0, a1, a2, a3,
              slab, idxs_v, idxd_v, rows_v, ones_v, zbuf_v, gsem, ssem):
    cid = lax.axis_index("c")
    sid = lax.axis_index("s")
    _fill_rows(ones_v, EBLK, 1.0)
    _fill_rows(zbuf_v, EBLK, 0.0)
    tables = (t0, t1, t2, t3)
    aggs = (a0, a1, a2, a3)
    degs = (deg0, deg1)

    for c in range(NC):
        @pl.when(cid == c)
        def _(c=c):
            _zero_slab(slab, zbuf_v, sid)
            plsc.subcore_barrier()
            _ones_pass(dst_hbm, slab, idxd_v, ones_v, ssem,
                       c * HALF_BLOCKS, BLK_PER_TILE_HALF, sid)
            plsc.subcore_barrier()
            _dump_slab(slab, degs[c], rows_v[0], sid)
            for q in (2 * c, 2 * c + 1):
                _zero_slab(slab, zbuf_v, sid)
                plsc.subcore_barrier()
                _gather_pass(src_hbm, dst_hbm, tables[q], slab,
                             idxs_v, idxd_v, rows_v, gsem, ssem, sid)
                plsc.subcore_barrier()
                _dump_slab(slab, aggs[q], rows_v[0], sid)


def _sc2_body(src_hbm, dst_hbm, t0, t1, t2, t3,
              a0, a1, a2, a3,
              slab, idxs_v, idxd_v, rows_v, zbuf_v, gsem, ssem):
    cid = lax.axis_index("c")
    sid = lax.axis_index("s")
    _fill_rows(zbuf_v, EBLK, 0.0)
    tables = (t0, t1, t2, t3)
    aggs = (a0, a1, a2, a3)

    for c in range(NC):
        @pl.when(cid == c)
        def _(c=c):
            for q in (2 * c, 2 * c + 1):
                _zero_slab(slab, zbuf_v, sid)
                plsc.subcore_barrier()
                _gather_pass(src_hbm, dst_hbm, tables[q], slab,
                             idxs_v, idxd_v, rows_v, gsem, ssem, sid)
                plsc.subcore_barrier()
                _dump_slab(slab, aggs[q], rows_v[0], sid)


def _sc_agg1(src2d, dst2d, hl0q):
    out = [jax.ShapeDtypeStruct((NPAD, FC), jnp.float32)] * 6
    scratch = [
        pltpu.VMEM_SHARED((NPAD, FC), jnp.float32),
        [pltpu.VMEM((NB, EBLK), jnp.int32)] * 2,
        [pltpu.VMEM((NB, EBLK), jnp.int32)] * 2,
        [pltpu.VMEM((NB * EBLK, FC), jnp.float32)] * 2,
        pltpu.VMEM((EBLK, FC), jnp.float32),
        pltpu.VMEM((EBLK, FC), jnp.float32),
        [pltpu.SemaphoreType.DMA] * 2,
        [pltpu.SemaphoreType.DMA] * 2,
    ]
    fn = pl.kernel(_sc1_body, out_type=out, mesh=_sc_mesh(),
                   scratch_types=scratch,
                   compiler_params=pltpu.CompilerParams(
                       use_tc_tiling_on_sc=False))
    return fn(src2d, dst2d, *hl0q)


def _sc_agg2(src2d, dst2d, h1q):
    out = [jax.ShapeDtypeStruct((NPAD, FC), jnp.float32)] * 4
    scratch = [
        pltpu.VMEM_SHARED((NPAD, FC), jnp.float32),
        [pltpu.VMEM((NB, EBLK), jnp.int32)] * 2,
        [pltpu.VMEM((NB, EBLK), jnp.int32)] * 2,
        [pltpu.VMEM((NB * EBLK, FC), jnp.float32)] * 2,
        pltpu.VMEM((EBLK, FC), jnp.float32),
        [pltpu.SemaphoreType.DMA] * 2,
        [pltpu.SemaphoreType.DMA] * 2,
    ]
    fn = pl.kernel(_sc2_body, out_type=out, mesh=_sc_mesh(),
                   scratch_types=scratch,
                   compiler_params=pltpu.CompilerParams(
                       use_tc_tiling_on_sc=False))
    return fn(src2d, dst2d, *h1q)


def _tc1_body(x_ref, emb_ref, wl_ref, wr_ref, hl_ref, hr_ref):
    xb = x_ref[0, 0, :]
    onehot = (xb[:, None] ==
              lax.broadcasted_iota(jnp.int32, (TCB, V), 1)
              ).astype(jnp.float32)
    tl = jnp.dot(emb_ref[...], wl_ref[...],
                 preferred_element_type=jnp.float32)
    tr = jnp.dot(emb_ref[...], wr_ref[...],
                 preferred_element_type=jnp.float32)
    hl_ref[...] = jnp.dot(onehot, tl, preferred_element_type=jnp.float32)
    hr_ref[...] = jnp.dot(onehot, tr, preferred_element_type=jnp.float32)


def _tc_prep(x3d, embed, W1l, W1r):
    return pl.pallas_call(
        _tc1_body,
        grid=(NBLK_TC,),
        in_specs=[
            pl.BlockSpec((1, 1, TCB), lambda i: (i, 0, 0)),
            pl.BlockSpec((V, D), lambda i: (0, 0)),
            pl.BlockSpec((D, H), lambda i: (0, 0)),
            pl.BlockSpec((D, H), lambda i: (0, 0)),
        ],
        out_specs=[pl.BlockSpec((TCB, H), lambda i: (i, 0))] * 2,
        out_shape=[jax.ShapeDtypeStruct((NPAD, H), jnp.float32)] * 2,
    )(x3d, embed, W1l, W1r)


def _tc2_body(a_ref, dg_ref, hr_ref, b1_ref, o_ref):
    deg = jnp.maximum(dg_ref[:, 0:1] + dg_ref[:, FC:FC + 1], 1.0)
    z = a_ref[...] * (1.0 / deg) + hr_ref[...] + b1_ref[...]
    o_ref[...] = jnp.maximum(z, 0.0)


def _tc_relu1(agg1, degs, hr0, b1_2d):
    return pl.pallas_call(
        _tc2_body,
        grid=(NBLK_TC,),
        in_specs=[
            pl.BlockSpec((TCB, H), lambda i: (i, 0)),
            pl.BlockSpec((TCB, NC * FC), lambda i: (i, 0)),
            pl.BlockSpec((TCB, H), lambda i: (i, 0)),
            pl.BlockSpec((1, H), lambda i: (0, 0)),
        ],
        out_specs=pl.BlockSpec((TCB, H), lambda i: (i, 0)),
        out_shape=jax.ShapeDtypeStruct((NPAD, H), jnp.float32),
    )(agg1, degs, hr0, b1_2d)


def _tc3_body(a_ref, h_ref, dg_ref, w2l_ref, w2r_ref, b2_ref, batch_ref,
              pooled_ref, cnt_ref):
    i = pl.program_id(0)

    @pl.when(i == 0)
    def _():
        pooled_ref[...] = jnp.zeros_like(pooled_ref)
        cnt_ref[...] = jnp.zeros_like(cnt_ref)

    deginv = 1.0 / jnp.maximum(dg_ref[:, 0:1] + dg_ref[:, FC:FC + 1], 1.0)
    agg = a_ref[...] * deginv
    z = (jnp.dot(agg, w2l_ref[...], preferred_element_type=jnp.float32)
         + jnp.dot(h_ref[...], w2r_ref[...],
                   preferred_element_type=jnp.float32)
         + b2_ref[...])
    h2 = jnp.maximum(z, 0.0)
    bb = batch_ref[0, 0, :]
    onehot_t = (lax.broadcasted_iota(jnp.int32, (G, TCB), 0) ==
                bb[None, :]).astype(jnp.float32)
    pooled_ref[...] += jnp.dot(onehot_t, h2,
                               preferred_element_type=jnp.float32)
    cnt_ref[...] += jnp.dot(onehot_t, jnp.ones((TCB, H), jnp.float32),
                            preferred_element_type=jnp.float32)


def _tc_layer2_pool(agg2, h1, degs, W2l, W2r, b2_2d, batch3d):
    return pl.pallas_call(
        _tc3_body,
        grid=(NBLK_TC,),
        in_specs=[
            pl.BlockSpec((TCB, H), lambda i: (i, 0)),
            pl.BlockSpec((TCB, H), lambda i: (i, 0)),
            pl.BlockSpec((TCB, NC * FC), lambda i: (i, 0)),
            pl.BlockSpec((H, H), lambda i: (0, 0)),
            pl.BlockSpec((H, H), lambda i: (0, 0)),
            pl.BlockSpec((1, H), lambda i: (0, 0)),
            pl.BlockSpec((1, 1, TCB), lambda i: (i, 0, 0)),
        ],
        out_specs=[
            pl.BlockSpec((G, H), lambda i: (0, 0)),
            pl.BlockSpec((G, H), lambda i: (0, 0)),
        ],
        out_shape=[
            jax.ShapeDtypeStruct((G, H), jnp.float32),
            jax.ShapeDtypeStruct((G, H), jnp.float32),
        ],
    )(agg2, h1, degs, W2l, W2r, b2_2d, batch3d)


def _tc4_body(pooled_ref, cnt_ref, wlin_ref, blin_ref, out_ref):
    pm = pooled_ref[...] / jnp.maximum(cnt_ref[...], 1.0)
    out_ref[...] = (jnp.dot(pm, wlin_ref[...],
                            preferred_element_type=jnp.float32)
                    + blin_ref[...])


def _tc_head(pooled, cnt, Wlin, blin_2d):
    return pl.pallas_call(
        _tc4_body,
        out_shape=jax.ShapeDtypeStruct((G, C), jnp.float32),
    )(pooled, cnt, Wlin, blin_2d)


def _chunks(arr):
    return [jnp.copy(arr[:, q * FC:(q + 1) * FC]) for q in range(NQ)]


@jax.jit
def _run(x, edge_index, batch, embed, W1l, W1r, b1, W2l, W2r, b2,
         Wlin, blin):
    x = x.astype(jnp.int32)
    batch = batch.astype(jnp.int32)
    src = edge_index[0].astype(jnp.int32)
    dst = edge_index[1].astype(jnp.int32)

    src2d = jnp.concatenate(
        [src, jnp.zeros((EPAD - E,), jnp.int32)]).reshape(EPAD_BLOCKS, EBLK)
    dst2d = jnp.concatenate(
        [dst, jnp.full((EPAD - E,), N, jnp.int32)]).reshape(EPAD_BLOCKS, EBLK)
    x3d = jnp.concatenate(
        [x, jnp.zeros((NPAD - N,), jnp.int32)]).reshape(NBLK_TC, 1, TCB)
    batch3d = jnp.concatenate(
        [batch, jnp.full((NPAD - N,), G, jnp.int32)]).reshape(NBLK_TC, 1, TCB)
    b1_2d = b1.reshape(1, H)
    b2_2d = b2.reshape(1, H)
    blin_2d = blin.reshape(1, C)

    hl0, hr0 = _tc_prep(x3d, embed, W1l, W1r)
    deg0, deg1, a0, a1, a2, a3 = _sc_agg1(src2d, dst2d, _chunks(hl0))
    degs = jnp.concatenate([deg0, deg1], axis=1)
    agg1 = jnp.concatenate([a0, a1, a2, a3], axis=1)
    h1 = _tc_relu1(agg1, degs, hr0, b1_2d)
    c0, c1, c2, c3 = _sc_agg2(src2d, dst2d, _chunks(h1))
    agg2 = jnp.concatenate([c0, c1, c2, c3], axis=1)
    pooled, cnt = _tc_layer2_pool(agg2, h1, degs, W2l, W2r,
                                  b2_2d, batch3d)
    return _tc_head(pooled, cnt, Wlin, blin_2d)


def kernel(x, edge_index, batch, embed, W1l, W1r, b1, W2l, W2r, b2,
           Wlin, blin):
    return _run(x, edge_index, batch, embed, W1l, W1r, b1, W2l, W2r, b2,
                Wlin, blin)
